# baseline (speedup 1.0000x reference)
import functools

import jax
import jax.numpy as jnp
from jax import lax
from jax.experimental import pallas as pl
from jax.experimental.pallas import tpu as pltpu
from jax.experimental.pallas import tpu_sc as plsc

IDX_H2 = 10

NC = 2
NS = 16
L = 16
NW = NC * NS
NSUB = 2

_LOG2_10 = 3.321928094887362
_S1 = 127.0 / (13.0 * _LOG2_10)
_A = _S1 / float(1 << 23)


def _build_tables(K, xt_v, xs_v, rdx_v, w_v, sc_v):
    zero = jnp.zeros((L,), jnp.int32)
    rc = plsc.load_gather(sc_v, [zero])
    bc = plsc.load_gather(sc_v, [zero + 1])
    for k in range(K // L):
        sl = pl.ds(k * L, L)
        xs_v[sl] = xt_v[sl] * rc
    for k in range(K // L):
        sl = pl.ds(k * L, L)
        x0 = xs_v[sl]
        if (k + 1) * L < K:
            x1 = xs_v[pl.ds(k * L + 1, L)]
            r = 1.0 / (x1 - x0)
        else:
            idx = jnp.minimum(lax.iota(jnp.int32, L) + (k * L + 1), K - 1)
            x1 = plsc.load_gather(xs_v, [idx])
            d = x1 - x0
            r = 1.0 / jnp.where(d == 0.0, 1.0, d)
        rdx_v[sl] = r
        w_v[sl] = -x0 * r
    return bc


def _interp_loop(K, steps, bc, avb, yb, ob, xs_v, rdx_v, w_v, fac_v):
    def step(i):
        sl = pl.ds(i * L, L)
        q = avb[sl] * yb[sl]
        bits = lax.bitcast_convert_type(q, jnp.int32)
        idx_f = jnp.clip(bits.astype(jnp.float32) * _A + bc,
                         0.0, float(K - 3))
        j = idx_f.astype(jnp.int32)
        jp = j + 1
        xm = plsc.load_gather(xs_v, [jp])
        i0 = jnp.where(q >= xm, jp, j)
        rdx0 = plsc.load_gather(rdx_v, [i0])
        w0 = plsc.load_gather(w_v, [i0])
        f0 = plsc.load_gather(fac_v, [i0])
        f1 = plsc.load_gather(fac_v, [i0 + 1])
        t = jnp.clip(q * rdx0 + w0, 0.0, 1.0)
        ob[sl] = f0 + (f1 - f0) * t

    plsc.parallel_loop(0, steps, 1, unroll=8)(step)


def _scratch_types(sub, K, extra):
    return [
        pltpu.VMEM((sub,), jnp.float32),
        pltpu.VMEM((sub,), jnp.float32),
        pltpu.VMEM((sub,), jnp.float32),
        pltpu.VMEM((sub,), jnp.float32),
        pltpu.VMEM((sub,), jnp.float32),
        pltpu.VMEM((sub,), jnp.float32),
        pltpu.VMEM((K,), jnp.float32),
        pltpu.VMEM((K,), jnp.float32),
        pltpu.VMEM((K,), jnp.float32),
        pltpu.VMEM((K,), jnp.float32),
        pltpu.VMEM((K,), jnp.float32),
        pltpu.VMEM((L,), jnp.float32),
        pltpu.SemaphoreType.DMA,
        pltpu.SemaphoreType.DMA,
        pltpu.SemaphoreType.DMA,
        pltpu.SemaphoreType.DMA,
        pltpu.SemaphoreType.DMA,
    ] + extra


def _make_half1(H, K):
    chunk = H // NW
    sub = chunk // NSUB
    steps = sub // L
    mesh = plsc.VectorSubcoreMesh(core_axis_name="c", subcore_axis_name="s",
                                  num_cores=NC, num_subcores=NS)

    @functools.partial(
        pl.kernel,
        out_type=jax.ShapeDtypeStruct((H,), jnp.float32),
        mesh=mesh,
        compiler_params=pltpu.CompilerParams(needs_layout_passes=False),
        scratch_types=_scratch_types(sub, K, []),
    )
    def sc_call(av_hbm, yc_hbm, xt_hbm, fac_hbm, sc_hbm, out_hbm,
                av0, av1, yc0, yc1, ot0, ot1,
                xt_v, xs_v, rdx_v, w_v, fac_v, sc_v,
                sem_in0, sem_in1, sem_out0, sem_out1, sem_t):
        wid = lax.axis_index("s") * NC + lax.axis_index("c")
        base = wid * chunk
        av_s, yc_s, out_s = (av0, av1), (yc0, yc1), (ot0, ot1)
        sems_in, sems_out = (sem_in0, sem_in1), (sem_out0, sem_out1)

        tcopies = [pltpu.async_copy(xt_hbm, xt_v, sem_t),
                   pltpu.async_copy(fac_hbm, fac_v, sem_t),
                   pltpu.async_copy(sc_hbm, sc_v, sem_t)]

        def start_in(g):
            s = g % 2
            lo = base + g * sub
            return (pltpu.async_copy(av_hbm.at[pl.ds(lo, sub)], av_s[s], sems_in[s]),
                    pltpu.async_copy(yc_hbm.at[pl.ds(lo, sub)], yc_s[s], sems_in[s]))

        pend_in = {0: start_in(0)}
        pend_out = {}
        for d in tcopies:
            d.wait()
        bc = _build_tables(K, xt_v, xs_v, rdx_v, w_v, sc_v)

        for g in range(NSUB):
            s = g % 2
            if g + 1 < NSUB:
                pend_in[g + 1] = start_in(g + 1)
            for d in pend_in.pop(g):
                d.wait()
            if g - 2 in pend_out:
                pend_out.pop(g - 2).wait()
            _interp_loop(K, steps, bc, av_s[s], yc_s[s], out_s[s],
                         xs_v, rdx_v, w_v, fac_v)
            pend_out[g] = pltpu.async_copy(
                out_s[s], out_hbm.at[pl.ds(base + g * sub, sub)], sems_out[s])
        for g in sorted(pend_out):
            pend_out.pop(g).wait()

    return sc_call


def _make_half2(B, H, K):
    chunk = (B - H) // NW
    sub = chunk // NSUB
    steps = sub // L
    mesh = plsc.VectorSubcoreMesh(core_axis_name="c", subcore_axis_name="s",
                                  num_cores=NC, num_subcores=NS)

    @functools.partial(
        pl.kernel,
        out_type=jax.ShapeDtypeStruct((B,), jnp.float32),
        mesh=mesh,
        compiler_params=pltpu.CompilerParams(needs_layout_passes=False),
        scratch_types=_scratch_types(sub, K, [
            pltpu.VMEM((chunk,), jnp.float32),
            pltpu.SemaphoreType.DMA,
        ]),
    )
    def sc_call(av_hbm, yc_hbm, xt_hbm, fac_hbm, sc_hbm, prev_hbm, out_hbm,
                av0, av1, yc0, yc1, ot0, ot1,
                xt_v, xs_v, rdx_v, w_v, fac_v, sc_v,
                sem_in0, sem_in1, sem_out0, sem_out1, sem_t,
                pass_v, sem_p):
        wid = lax.axis_index("s") * NC + lax.axis_index("c")
        base = wid * chunk
        av_s, yc_s, out_s = (av0, av1), (yc0, yc1), (ot0, ot1)
        sems_in, sems_out = (sem_in0, sem_in1), (sem_out0, sem_out1)

        p_in = pltpu.async_copy(prev_hbm.at[pl.ds(base, chunk)], pass_v, sem_p)
        tcopies = [pltpu.async_copy(xt_hbm, xt_v, sem_t),
                   pltpu.async_copy(fac_hbm, fac_v, sem_t),
                   pltpu.async_copy(sc_hbm, sc_v, sem_t)]

        def start_in(g):
            s = g % 2
            lo = base + g * sub
            return (pltpu.async_copy(av_hbm.at[pl.ds(H + lo, sub)], av_s[s], sems_in[s]),
                    pltpu.async_copy(yc_hbm.at[pl.ds(lo, sub)], yc_s[s], sems_in[s]))

        pend_in = {0: start_in(0)}
        pend_out = {}
        for d in tcopies:
            d.wait()
        bc = _build_tables(K, xt_v, xs_v, rdx_v, w_v, sc_v)
        p_in.wait()
        p_out = pltpu.async_copy(pass_v, out_hbm.at[pl.ds(base, chunk)], sem_p)

        for g in range(NSUB):
            s = g % 2
            if g + 1 < NSUB:
                pend_in[g + 1] = start_in(g + 1)
            for d in pend_in.pop(g):
                d.wait()
            if g - 2 in pend_out:
                pend_out.pop(g - 2).wait()
            _interp_loop(K, steps, bc, av_s[s], yc_s[s], out_s[s],
                         xs_v, rdx_v, w_v, fac_v)
            pend_out[g] = pltpu.async_copy(
                out_s[s], out_hbm.at[pl.ds(H + base + g * sub, sub)],
                sems_out[s])
        for g in sorted(pend_out):
            pend_out.pop(g).wait()
        p_out.wait()

    return sc_call


def kernel(Av, params_reac, y_in, x_H2, factor, den_Av_ratio_0):
    B = Av.shape[0]
    K = x_H2.shape[0]
    H = B // 2
    av = Av.reshape(B)
    yc1 = y_in[:H, IDX_H2]
    yc2 = y_in[H:, IDX_H2]
    fac = factor.reshape(K)
    c = den_Av_ratio_0.astype(jnp.float32)
    bc = (-_S1 * (127.0 + jnp.log2(x_H2[0] / c))).astype(jnp.float32)
    scal = jnp.tile(jnp.stack([1.0 / c, bc]), L // 2)
    out1 = _make_half1(H, K)(av, yc1, x_H2, fac, scal)
    out = _make_half2(B, H, K)(av, yc2, x_H2, fac, scal, out1)
    return out.reshape(B, 1)


# speedup vs baseline: 1.0677x; 1.0677x over previous
import functools

import jax
import jax.numpy as jnp
from jax import lax
from jax.experimental import pallas as pl
from jax.experimental.pallas import tpu as pltpu
from jax.experimental.pallas import tpu_sc as plsc

IDX_H2 = 10

NC = 2
NS = 16
L = 16
NW = NC * NS
NSUB = 2

_LOG2_10 = 3.321928094887362
_S1 = 127.0 / (13.0 * _LOG2_10)
_A = _S1 / float(1 << 23)


def _build_tables(K, xt_v, xs_v, rdx_v, w_v, sc_v):
    zero = jnp.zeros((L,), jnp.int32)
    rc = plsc.load_gather(sc_v, [zero])
    bc = plsc.load_gather(sc_v, [zero + 1])
    for k in range(K // L):
        sl = pl.ds(k * L, L)
        xs_v[sl] = xt_v[sl] * rc
    for k in range(K // L):
        sl = pl.ds(k * L, L)
        x0 = xs_v[sl]
        if (k + 1) * L < K:
            x1 = xs_v[pl.ds(k * L + 1, L)]
            r = 1.0 / (x1 - x0)
        else:
            idx = jnp.minimum(lax.iota(jnp.int32, L) + (k * L + 1), K - 1)
            x1 = plsc.load_gather(xs_v, [idx])
            d = x1 - x0
            r = 1.0 / jnp.where(d == 0.0, 1.0, d)
        rdx_v[sl] = r
        w_v[sl] = -x0 * r
    return bc


def _interp_loop(K, steps, bc, avb, yb, ob, xs_v, rdx_v, w_v, fac_v):
    def step(i):
        sl = pl.ds(i * L, L)
        q = avb[sl] * yb[sl]
        bits = lax.bitcast_convert_type(q, jnp.int32)
        idx_f = jnp.clip(bits.astype(jnp.float32) * _A + bc,
                         0.0, float(K - 3))
        j = idx_f.astype(jnp.int32)
        jp = j + 1
        xm = plsc.load_gather(xs_v, [jp])
        i0 = jnp.where(q >= xm, jp, j)
        rdx0 = plsc.load_gather(rdx_v, [i0])
        w0 = plsc.load_gather(w_v, [i0])
        f0 = plsc.load_gather(fac_v, [i0])
        f1 = plsc.load_gather(fac_v, [i0 + 1])
        t = jnp.clip(q * rdx0 + w0, 0.0, 1.0)
        ob[sl] = f0 + (f1 - f0) * t

    plsc.parallel_loop(0, steps, 1, unroll=8)(step)


def _scratch_types(sub, K, extra):
    return [
        pltpu.VMEM((sub,), jnp.float32),
        pltpu.VMEM((sub,), jnp.float32),
        pltpu.VMEM((sub,), jnp.float32),
        pltpu.VMEM((sub,), jnp.float32),
        pltpu.VMEM((sub,), jnp.float32),
        pltpu.VMEM((sub,), jnp.float32),
        pltpu.VMEM((K,), jnp.float32),
        pltpu.VMEM((K,), jnp.float32),
        pltpu.VMEM((K,), jnp.float32),
        pltpu.VMEM((K,), jnp.float32),
        pltpu.VMEM((K,), jnp.float32),
        pltpu.VMEM((L,), jnp.float32),
        pltpu.SemaphoreType.DMA,
        pltpu.SemaphoreType.DMA,
        pltpu.SemaphoreType.DMA,
        pltpu.SemaphoreType.DMA,
        pltpu.SemaphoreType.DMA,
    ] + extra


def _make_half1(H, K):
    chunk = H // NW
    sub = chunk // NSUB
    steps = sub // L
    mesh = plsc.VectorSubcoreMesh(core_axis_name="c", subcore_axis_name="s",
                                  num_cores=NC, num_subcores=NS)

    @functools.partial(
        pl.kernel,
        out_type=jax.ShapeDtypeStruct((H,), jnp.float32),
        mesh=mesh,
        compiler_params=pltpu.CompilerParams(needs_layout_passes=False),
        scratch_types=_scratch_types(sub, K, []),
    )
    def sc_call(av_hbm, yc_hbm, xt_hbm, fac_hbm, sc_hbm, out_hbm,
                av0, av1, yc0, yc1, ot0, ot1,
                xt_v, xs_v, rdx_v, w_v, fac_v, sc_v,
                sem_in0, sem_in1, sem_out0, sem_out1, sem_t):
        wid = lax.axis_index("s") * NC + lax.axis_index("c")
        base = wid * chunk
        av_s, yc_s, out_s = (av0, av1), (yc0, yc1), (ot0, ot1)
        sems_in, sems_out = (sem_in0, sem_in1), (sem_out0, sem_out1)

        tcopies = [pltpu.async_copy(xt_hbm, xt_v, sem_t),
                   pltpu.async_copy(fac_hbm, fac_v, sem_t),
                   pltpu.async_copy(sc_hbm, sc_v, sem_t)]

        def start_in(g):
            s = g % 2
            lo = base + g * sub
            return (pltpu.async_copy(av_hbm.at[pl.ds(lo, sub)], av_s[s], sems_in[s]),
                    pltpu.async_copy(yc_hbm.at[pl.ds(lo, sub)], yc_s[s], sems_in[s]))

        pend_in = {0: start_in(0)}
        pend_out = {}
        for d in tcopies:
            d.wait()
        bc = _build_tables(K, xt_v, xs_v, rdx_v, w_v, sc_v)

        for g in range(NSUB):
            s = g % 2
            if g + 1 < NSUB:
                pend_in[g + 1] = start_in(g + 1)
            for d in pend_in.pop(g):
                d.wait()
            if g - 2 in pend_out:
                pend_out.pop(g - 2).wait()
            _interp_loop(K, steps, bc, av_s[s], yc_s[s], out_s[s],
                         xs_v, rdx_v, w_v, fac_v)
            pend_out[g] = pltpu.async_copy(
                out_s[s], out_hbm.at[pl.ds(base + g * sub, sub)], sems_out[s])
        for g in sorted(pend_out):
            pend_out.pop(g).wait()

    return sc_call


def _make_half2(B, H, K):
    chunk = (B - H) // NW
    sub = chunk // NSUB
    steps = sub // L
    mesh = plsc.VectorSubcoreMesh(core_axis_name="c", subcore_axis_name="s",
                                  num_cores=NC, num_subcores=NS)

    @functools.partial(
        pl.kernel,
        out_type=jax.ShapeDtypeStruct((B,), jnp.float32),
        mesh=mesh,
        compiler_params=pltpu.CompilerParams(needs_layout_passes=False),
        scratch_types=_scratch_types(sub, K, [
            pltpu.VMEM((chunk,), jnp.float32),
            pltpu.SemaphoreType.DMA,
        ]),
    )
    def sc_call(av_hbm, yc_hbm, xt_hbm, fac_hbm, sc_hbm, prev_hbm, out_hbm,
                av0, av1, yc0, yc1, ot0, ot1,
                xt_v, xs_v, rdx_v, w_v, fac_v, sc_v,
                sem_in0, sem_in1, sem_out0, sem_out1, sem_t,
                pass_v, sem_p):
        wid = lax.axis_index("s") * NC + lax.axis_index("c")
        base = wid * chunk
        av_s, yc_s, out_s = (av0, av1), (yc0, yc1), (ot0, ot1)
        sems_in, sems_out = (sem_in0, sem_in1), (sem_out0, sem_out1)

        p_in = pltpu.async_copy(prev_hbm.at[pl.ds(base, chunk)], pass_v, sem_p)
        tcopies = [pltpu.async_copy(xt_hbm, xt_v, sem_t),
                   pltpu.async_copy(fac_hbm, fac_v, sem_t),
                   pltpu.async_copy(sc_hbm, sc_v, sem_t)]

        def start_in(g):
            s = g % 2
            lo = base + g * sub
            return (pltpu.async_copy(av_hbm.at[pl.ds(H + lo, sub)], av_s[s], sems_in[s]),
                    pltpu.async_copy(yc_hbm.at[pl.ds(lo, sub)], yc_s[s], sems_in[s]))

        pend_in = {0: start_in(0)}
        pend_out = {}
        for d in tcopies:
            d.wait()
        bc = _build_tables(K, xt_v, xs_v, rdx_v, w_v, sc_v)
        p_in.wait()
        p_out = pltpu.async_copy(pass_v, out_hbm.at[pl.ds(base, chunk)], sem_p)

        for g in range(NSUB):
            s = g % 2
            if g + 1 < NSUB:
                pend_in[g + 1] = start_in(g + 1)
            for d in pend_in.pop(g):
                d.wait()
            if g - 2 in pend_out:
                pend_out.pop(g - 2).wait()
            _interp_loop(K, steps, bc, av_s[s], yc_s[s], out_s[s],
                         xs_v, rdx_v, w_v, fac_v)
            pend_out[g] = pltpu.async_copy(
                out_s[s], out_hbm.at[pl.ds(H + base + g * sub, sub)],
                sems_out[s])
        for g in sorted(pend_out):
            pend_out.pop(g).wait()
        p_out.wait()

    return sc_call


def kernel(Av, params_reac, y_in, x_H2, factor, den_Av_ratio_0):
    B = Av.shape[0]
    K = x_H2.shape[0]
    H = B // 2
    av = Av.reshape(B)
    yc1 = y_in[:H, IDX_H2]
    yc1, y_in_b = lax.optimization_barrier((yc1, y_in))
    yc2 = y_in_b[H:, IDX_H2]
    fac = factor.reshape(K)
    c = den_Av_ratio_0.astype(jnp.float32)
    bc = (-_S1 * (127.0 + jnp.log2(x_H2[0] / c))).astype(jnp.float32)
    scal = jnp.tile(jnp.stack([1.0 / c, bc]), L // 2)
    out1 = _make_half1(H, K)(av, yc1, x_H2, fac, scal)
    out = _make_half2(B, H, K)(av, yc2, x_H2, fac, scal, out1)
    return out.reshape(B, 1)


# speedup vs baseline: 1.0719x; 1.0039x over previous
import functools

import jax
import jax.numpy as jnp
from jax import lax
from jax.experimental import pallas as pl
from jax.experimental.pallas import tpu as pltpu
from jax.experimental.pallas import tpu_sc as plsc

IDX_H2 = 10

NC = 2
NS = 16
L = 16
NW = NC * NS
NSUB = 2

_LOG2_10 = 3.321928094887362
_S1 = 127.0 / (13.0 * _LOG2_10)
_A = _S1 / float(1 << 23)


def _build_tables(K, xt_v, xs_v, rdx_v, w_v, sc_v):
    zero = jnp.zeros((L,), jnp.int32)
    rc = plsc.load_gather(sc_v, [zero])
    bc = plsc.load_gather(sc_v, [zero + 1])
    for k in range(K // L):
        sl = pl.ds(k * L, L)
        xs_v[sl] = xt_v[sl] * rc
    for k in range(K // L):
        sl = pl.ds(k * L, L)
        x0 = xs_v[sl]
        if (k + 1) * L < K:
            x1 = xs_v[pl.ds(k * L + 1, L)]
            r = 1.0 / (x1 - x0)
        else:
            idx = jnp.minimum(lax.iota(jnp.int32, L) + (k * L + 1), K - 1)
            x1 = plsc.load_gather(xs_v, [idx])
            d = x1 - x0
            r = 1.0 / jnp.where(d == 0.0, 1.0, d)
        rdx_v[sl] = r
        w_v[sl] = -x0 * r
    return bc


def _interp_loop(K, steps, bc, avb, yb, ob, xs_v, rdx_v, w_v, fac_v):
    def step(i):
        sl = pl.ds(i * L, L)
        q = avb[sl] * yb[sl]
        bits = lax.bitcast_convert_type(q, jnp.int32)
        idx_f = jnp.clip(bits.astype(jnp.float32) * _A + bc,
                         0.0, float(K - 3))
        j = idx_f.astype(jnp.int32)
        jp = j + 1
        xm = plsc.load_gather(xs_v, [jp])
        i0 = jnp.where(q >= xm, jp, j)
        rdx0 = plsc.load_gather(rdx_v, [i0])
        w0 = plsc.load_gather(w_v, [i0])
        f0 = plsc.load_gather(fac_v, [i0])
        f1 = plsc.load_gather(fac_v, [i0 + 1])
        t = jnp.clip(q * rdx0 + w0, 0.0, 1.0)
        ob[sl] = f0 + (f1 - f0) * t

    plsc.parallel_loop(0, steps, 1, unroll=8)(step)


def _scratch_types(sub, K, extra):
    return [
        pltpu.VMEM((sub,), jnp.float32),
        pltpu.VMEM((sub,), jnp.float32),
        pltpu.VMEM((sub,), jnp.float32),
        pltpu.VMEM((sub,), jnp.float32),
        pltpu.VMEM((sub,), jnp.float32),
        pltpu.VMEM((sub,), jnp.float32),
        pltpu.VMEM((K,), jnp.float32),
        pltpu.VMEM((K,), jnp.float32),
        pltpu.VMEM((K,), jnp.float32),
        pltpu.VMEM((K,), jnp.float32),
        pltpu.VMEM((K,), jnp.float32),
        pltpu.VMEM((L,), jnp.float32),
        pltpu.SemaphoreType.DMA,
        pltpu.SemaphoreType.DMA,
        pltpu.SemaphoreType.DMA,
        pltpu.SemaphoreType.DMA,
        pltpu.SemaphoreType.DMA,
    ] + extra


def _make_half1(H, K):
    chunk = H // NW
    sub = chunk // NSUB
    steps = sub // L
    mesh = plsc.VectorSubcoreMesh(core_axis_name="c", subcore_axis_name="s",
                                  num_cores=NC, num_subcores=NS)

    @functools.partial(
        pl.kernel,
        out_type=jax.ShapeDtypeStruct((H,), jnp.float32),
        mesh=mesh,
        compiler_params=pltpu.CompilerParams(needs_layout_passes=False),
        scratch_types=_scratch_types(sub, K, []),
    )
    def sc_call(av_hbm, yc_hbm, xt_hbm, fac_hbm, sc_hbm, out_hbm,
                av0, av1, yc0, yc1, ot0, ot1,
                xt_v, xs_v, rdx_v, w_v, fac_v, sc_v,
                sem_in0, sem_in1, sem_out0, sem_out1, sem_t):
        wid = lax.axis_index("s") * NC + lax.axis_index("c")
        base = wid * chunk
        av_s, yc_s, out_s = (av0, av1), (yc0, yc1), (ot0, ot1)
        sems_in, sems_out = (sem_in0, sem_in1), (sem_out0, sem_out1)

        tcopies = [pltpu.async_copy(xt_hbm, xt_v, sem_t),
                   pltpu.async_copy(fac_hbm, fac_v, sem_t),
                   pltpu.async_copy(sc_hbm, sc_v, sem_t)]

        def start_in(g):
            s = g % 2
            lo = base + g * sub
            return (pltpu.async_copy(av_hbm.at[pl.ds(lo, sub)], av_s[s], sems_in[s]),
                    pltpu.async_copy(yc_hbm.at[pl.ds(lo, sub)], yc_s[s], sems_in[s]))

        pend_in = {0: start_in(0)}
        pend_out = {}
        for d in tcopies:
            d.wait()
        bc = _build_tables(K, xt_v, xs_v, rdx_v, w_v, sc_v)

        for g in range(NSUB):
            s = g % 2
            if g + 1 < NSUB:
                pend_in[g + 1] = start_in(g + 1)
            for d in pend_in.pop(g):
                d.wait()
            if g - 2 in pend_out:
                pend_out.pop(g - 2).wait()
            _interp_loop(K, steps, bc, av_s[s], yc_s[s], out_s[s],
                         xs_v, rdx_v, w_v, fac_v)
            pend_out[g] = pltpu.async_copy(
                out_s[s], out_hbm.at[pl.ds(base + g * sub, sub)], sems_out[s])
        for g in sorted(pend_out):
            pend_out.pop(g).wait()

    return sc_call


def _make_half2(B, H, K):
    chunk = (B - H) // NW
    sub = chunk // NSUB
    steps = sub // L
    mesh = plsc.VectorSubcoreMesh(core_axis_name="c", subcore_axis_name="s",
                                  num_cores=NC, num_subcores=NS)

    @functools.partial(
        pl.kernel,
        out_type=jax.ShapeDtypeStruct((B,), jnp.float32),
        mesh=mesh,
        compiler_params=pltpu.CompilerParams(needs_layout_passes=False),
        scratch_types=_scratch_types(sub, K, [
            pltpu.VMEM((chunk,), jnp.float32),
            pltpu.SemaphoreType.DMA,
        ]),
    )
    def sc_call(av_hbm, yc_hbm, xt_hbm, fac_hbm, sc_hbm, prev_hbm, out_hbm,
                av0, av1, yc0, yc1, ot0, ot1,
                xt_v, xs_v, rdx_v, w_v, fac_v, sc_v,
                sem_in0, sem_in1, sem_out0, sem_out1, sem_t,
                pass_v, sem_p):
        wid = lax.axis_index("s") * NC + lax.axis_index("c")
        base = wid * chunk
        av_s, yc_s, out_s = (av0, av1), (yc0, yc1), (ot0, ot1)
        sems_in, sems_out = (sem_in0, sem_in1), (sem_out0, sem_out1)

        p_in = pltpu.async_copy(prev_hbm.at[pl.ds(base, chunk)], pass_v, sem_p)
        tcopies = [pltpu.async_copy(xt_hbm, xt_v, sem_t),
                   pltpu.async_copy(fac_hbm, fac_v, sem_t),
                   pltpu.async_copy(sc_hbm, sc_v, sem_t)]

        def start_in(g):
            s = g % 2
            lo = base + g * sub
            return (pltpu.async_copy(av_hbm.at[pl.ds(H + lo, sub)], av_s[s], sems_in[s]),
                    pltpu.async_copy(yc_hbm.at[pl.ds(lo, sub)], yc_s[s], sems_in[s]))

        pend_in = {0: start_in(0)}
        pend_out = {}
        for d in tcopies:
            d.wait()
        bc = _build_tables(K, xt_v, xs_v, rdx_v, w_v, sc_v)
        p_in.wait()
        p_out = pltpu.async_copy(pass_v, out_hbm.at[pl.ds(base, chunk)], sem_p)

        for g in range(NSUB):
            s = g % 2
            if g + 1 < NSUB:
                pend_in[g + 1] = start_in(g + 1)
            for d in pend_in.pop(g):
                d.wait()
            if g - 2 in pend_out:
                pend_out.pop(g - 2).wait()
            _interp_loop(K, steps, bc, av_s[s], yc_s[s], out_s[s],
                         xs_v, rdx_v, w_v, fac_v)
            pend_out[g] = pltpu.async_copy(
                out_s[s], out_hbm.at[pl.ds(H + base + g * sub, sub)],
                sems_out[s])
        for g in sorted(pend_out):
            pend_out.pop(g).wait()
        p_out.wait()

    return sc_call


def kernel(Av, params_reac, y_in, x_H2, factor, den_Av_ratio_0):
    B = Av.shape[0]
    K = x_H2.shape[0]
    H = (2 * B // 5) // 1024 * 1024
    av = Av.reshape(B)
    yc1 = y_in[:H, IDX_H2]
    yc1, y_in_b = lax.optimization_barrier((yc1, y_in))
    yc2 = y_in_b[H:, IDX_H2]
    fac = factor.reshape(K)
    c = den_Av_ratio_0.astype(jnp.float32)
    bc = (-_S1 * (127.0 + jnp.log2(x_H2[0] / c))).astype(jnp.float32)
    scal = jnp.tile(jnp.stack([1.0 / c, bc]), L // 2)
    out1 = _make_half1(H, K)(av, yc1, x_H2, fac, scal)
    out = _make_half2(B, H, K)(av, yc2, x_H2, fac, scal, out1)
    return out.reshape(B, 1)


# speedup vs baseline: 1.1466x; 1.0697x over previous
import functools

import jax
import jax.numpy as jnp
from jax import lax
from jax.experimental import pallas as pl
from jax.experimental.pallas import tpu as pltpu
from jax.experimental.pallas import tpu_sc as plsc

IDX_H2 = 10

NC = 2
NS = 16
L = 16
NW = NC * NS
NSUB = 4

_LOG2_10 = 3.321928094887362
_S1 = 127.0 / (13.0 * _LOG2_10)
_A = _S1 / float(1 << 23)


def _make_sc_call(B, K):
    chunk = B // NW
    sub = chunk // NSUB
    steps = sub // L
    mesh = plsc.VectorSubcoreMesh(core_axis_name="c", subcore_axis_name="s",
                                  num_cores=NC, num_subcores=NS)

    @functools.partial(
        pl.kernel,
        out_type=jax.ShapeDtypeStruct((B,), jnp.float32),
        mesh=mesh,
        compiler_params=pltpu.CompilerParams(needs_layout_passes=False),
        scratch_types=[
            pltpu.VMEM((sub,), jnp.float32),
            pltpu.VMEM((sub,), jnp.float32),
            pltpu.VMEM((sub,), jnp.float32),
            pltpu.VMEM((sub,), jnp.float32),
            pltpu.VMEM((sub,), jnp.float32),
            pltpu.VMEM((sub,), jnp.float32),
            pltpu.VMEM((K,), jnp.float32),
            pltpu.VMEM((K,), jnp.float32),
            pltpu.VMEM((K,), jnp.float32),
            pltpu.VMEM((K,), jnp.float32),
            pltpu.VMEM((K,), jnp.float32),
            pltpu.VMEM((L,), jnp.float32),
            pltpu.SemaphoreType.DMA,
            pltpu.SemaphoreType.DMA,
            pltpu.SemaphoreType.DMA,
            pltpu.SemaphoreType.DMA,
            pltpu.SemaphoreType.DMA,
        ],
    )
    def sc_call(av_hbm, yc_hbm, xt_hbm, fac_hbm, sc_hbm, out_hbm,
                av0, av1, yc0, yc1, ot0, ot1,
                xt_v, xs_v, rdx_v, w_v, fac_v, sc_v,
                sem_in0, sem_in1, sem_out0, sem_out1, sem_t):
        wid = lax.axis_index("s") * NC + lax.axis_index("c")
        base = wid * chunk
        av_s = (av0, av1)
        yc_s = (yc0, yc1)
        out_s = (ot0, ot1)
        sems_in = (sem_in0, sem_in1)
        sems_out = (sem_out0, sem_out1)

        tcopies = [
            pltpu.async_copy(xt_hbm, xt_v, sem_t),
            pltpu.async_copy(fac_hbm, fac_v, sem_t),
            pltpu.async_copy(sc_hbm, sc_v, sem_t),
        ]

        def start_in(g):
            s = g % 2
            lo = base + g * sub
            a = pltpu.async_copy(av_hbm.at[pl.ds(lo, sub)], av_s[s], sems_in[s])
            y = pltpu.async_copy(yc_hbm.at[pl.ds(lo, sub)], yc_s[s], sems_in[s])
            return (a, y)

        pend_in = {0: start_in(0)}
        pend_out = {}
        for d in tcopies:
            d.wait()
        zero = jnp.zeros((L,), jnp.int32)
        rc = plsc.load_gather(sc_v, [zero])
        bc = plsc.load_gather(sc_v, [zero + 1])
        for k in range(K // L):
            sl = pl.ds(k * L, L)
            xs_v[sl] = xt_v[sl] * rc
        for k in range(K // L):
            sl = pl.ds(k * L, L)
            x0 = xs_v[sl]
            if (k + 1) * L < K:
                x1 = xs_v[pl.ds(k * L + 1, L)]
                r = 1.0 / (x1 - x0)
            else:
                idx = jnp.minimum(lax.iota(jnp.int32, L) + (k * L + 1), K - 1)
                x1 = plsc.load_gather(xs_v, [idx])
                d = x1 - x0
                r = 1.0 / jnp.where(d == 0.0, 1.0, d)
            rdx_v[sl] = r
            w_v[sl] = -x0 * r

        for g in range(NSUB):
            s = g % 2
            if g + 1 < NSUB:
                pend_in[g + 1] = start_in(g + 1)
            for d in pend_in.pop(g):
                d.wait()
            if g - 2 in pend_out:
                pend_out.pop(g - 2).wait()
            avb, yb, ob = av_s[s], yc_s[s], out_s[s]

            def step(i, avb=avb, yb=yb, ob=ob):
                sl = pl.ds(i * L, L)
                q = avb[sl] * yb[sl]
                bits = lax.bitcast_convert_type(q, jnp.int32)
                idx_f = jnp.clip(bits.astype(jnp.float32) * _A + bc,
                                 0.0, float(K - 3))
                j = idx_f.astype(jnp.int32)
                jp = j + 1
                xm = plsc.load_gather(xs_v, [jp])
                i0 = jnp.where(q >= xm, jp, j)
                rdx0 = plsc.load_gather(rdx_v, [i0])
                w0 = plsc.load_gather(w_v, [i0])
                f0 = plsc.load_gather(fac_v, [i0])
                f1 = plsc.load_gather(fac_v, [i0 + 1])
                t = jnp.clip(q * rdx0 + w0, 0.0, 1.0)
                ob[sl] = f0 + (f1 - f0) * t

            plsc.parallel_loop(0, steps, 1, unroll=8)(step)
            pend_out[g] = pltpu.async_copy(
                ob, out_hbm.at[pl.ds(base + g * sub, sub)], sems_out[s])
        for g in sorted(pend_out):
            pend_out.pop(g).wait()

    return sc_call


def kernel(Av, params_reac, y_in, x_H2, factor, den_Av_ratio_0):
    B = Av.shape[0]
    K = x_H2.shape[0]
    av = Av.reshape(B)
    yc = y_in[:, IDX_H2]
    fac = factor.reshape(K)
    c = den_Av_ratio_0.astype(jnp.float32)
    bc = (-_S1 * (127.0 + jnp.log2(x_H2[0] / c))).astype(jnp.float32)
    scal = jnp.tile(jnp.stack([1.0 / c, bc]), L // 2)
    out = _make_sc_call(B, K)(av, yc, x_H2, fac, scal)
    return out.reshape(B, 1)
